# Initial kernel scaffold; baseline (speedup 1.0000x reference)
import jax
import jax.numpy as jnp
from jax.experimental import pallas as pl


def kernel(dist_edges, dist_vec, batch, poi, x, poi_table, gcn0_w, gcn0_b, gcn1_w, gcn1_b, K_w, K_b, Q_w, Q_b, V_w, V_b, proj1_w, proj1_b, proj2_w, proj2_b, pred1_w, pred1_b, pred2_w, pred2_b):
    raise NotImplementedError("write your pallas kernel here")


# speedup vs baseline: 1.0021x; 1.0021x over previous
import functools

import jax
import jax.numpy as jnp
from jax.experimental import pallas as pl
from jax.experimental.pallas import tpu as pltpu

N_POI = 10000
E = 160000
D = 128
B = 1024
N_ITEMS = 20480


def _heads_body(aggr_ref, tar_ref, graph_ref,
                proj1_w_ref, proj1_b_ref, proj2_w_ref, proj2_b_ref,
                pred1_w_ref, pred1_b_ref, pred2_w_ref, pred2_b_ref,
                out1_ref, out2_ref):
    aggr = aggr_ref[...]
    tar = tar_ref[...]
    graph_enc = graph_ref[...]
    pred_in = jnp.concatenate([aggr, tar], axis=-1)
    h = pred_in @ pred1_w_ref[...].T + pred1_b_ref[...]
    h = jnp.where(h > 0, h, 0.01 * h)
    out2_ref[...] = h @ pred2_w_ref[...].T + pred2_b_ref[...]
    gh = graph_enc @ proj1_w_ref[...].T + proj1_b_ref[...]
    gh = jnp.where(gh > 0, gh, 0.01 * gh)
    out1_ref[...] = gh @ proj2_w_ref[...].T + proj2_b_ref[...]
    gh = graph_enc @ proj1_w_ref[...].T + proj1_b_ref[...]
    gh = jnp.where(gh > 0, gh, 0.01 * gh)
    out1_ref[...] = gh @ proj2_w_ref[...].T + proj2_b_ref[...]


def kernel(dist_edges, dist_vec, batch, poi, x, poi_table, gcn0_w, gcn0_b,
           gcn1_w, gcn1_b, K_w, K_b, Q_w, Q_b, V_w, V_b, proj1_w, proj1_b,
           proj2_w, proj2_b, pred1_w, pred1_b, pred2_w, pred2_b):
    loop = jnp.arange(N_POI)
    row = jnp.concatenate([dist_edges[0], dist_edges[1], loop])
    col = jnp.concatenate([dist_edges[1], dist_edges[0], loop])
    dv = jnp.concatenate([dist_vec, dist_vec, jnp.zeros(N_POI, jnp.float32)])
    deg = jax.ops.segment_sum(jnp.ones_like(col, dtype=jnp.float32), col,
                              num_segments=N_POI)
    dis = jnp.where(deg > 0, deg ** -0.5, 0.0)
    norm = dis[row] * dis[col]
    vals = jnp.exp(-dv ** 2) * norm
    enc = poi_table
    for w, b in ((gcn0_w, gcn0_b), (gcn1_w, gcn1_b)):
        side = jax.ops.segment_sum(vals[:, None] * enc[col], row,
                                   num_segments=N_POI)
        enc = side @ w.T + b
        enc = jax.nn.leaky_relu(enc)
        enc = enc / jnp.maximum(jnp.linalg.norm(enc, axis=-1, keepdims=True),
                                1e-12)
    tar = enc[poi]
    geo = enc[x]
    seq_lens = jnp.bincount(batch, length=B)
    v = geo @ K_w.T + K_b
    q = tar @ Q_w.T + Q_b
    aw_logits = jnp.sum(v * q[batch], axis=-1)
    seg_max = jax.ops.segment_max(aw_logits, batch, num_segments=B)
    aw_exp = jnp.exp(aw_logits - seg_max[batch])
    aw_den = jax.ops.segment_sum(aw_exp, batch, num_segments=B)
    aw = aw_exp / aw_den[batch]
    seq_feat = jax.ops.segment_sum(v * aw[:, None], batch, num_segments=B)
    aggr = seq_feat @ V_w.T + V_b
    graph_enc = (jax.ops.segment_sum(geo, batch, num_segments=B)
                 / seq_lens[:, None].astype(jnp.float32))

    pred2_w_pad = jnp.pad(pred2_w, ((0, D - 1), (0, 0)))
    pred2_b_pad = jnp.pad(pred2_b, (0, D - 1))
    out1, out2_pad = pl.pallas_call(
        _heads_body,
        out_shape=(jax.ShapeDtypeStruct((B, D), jnp.float32),
                   jax.ShapeDtypeStruct((B, D), jnp.float32)),
    )(aggr, tar, graph_enc, proj1_w, proj1_b, proj2_w, proj2_b,
      pred1_w, pred1_b, pred2_w_pad, pred2_b_pad)
    return (out1, out2_pad[:, :1])


# speedup vs baseline: 6.1363x; 6.1236x over previous
import functools

import jax
import jax.numpy as jnp
from jax import lax
from jax.experimental import pallas as pl
from jax.experimental.pallas import tpu as pltpu
from jax.experimental.pallas import tpu_sc as plsc

N_POI = 10000
E = 160000
D = 128
B = 1024
N_ITEMS = 20480

_NC, _NS, _L = 2, 16, 16
_NW = _NC * _NS
_NP = 10240
_RPT = _NP // _NS
_EK = 80
_EPW = 2 * E // _NW
_NCH = _EPW // _EK
_VGP = 313
_E_PAD = _VGP * 16 * _NW


def _mesh():
    return plsc.VectorSubcoreMesh(core_axis_name="c", subcore_axis_name="s",
                                  num_cores=_NC, num_subcores=_NS)


def _deg_body(flat_hbm, out_hbm, idx_v, ones_v, out_v, acc_sh):
    c = lax.axis_index("c")
    s = lax.axis_index("s")
    wid = c * _NS + s
    zeros16 = jnp.zeros((_L,), jnp.float32)

    def _fill(j, carry):
        ones_v[pl.ds(j * _L, _L)] = zeros16 + 1.0
        out_v[pl.ds(j * _L, _L)] = zeros16
        return carry
    lax.fori_loop(0, _RPT // _L, _fill, 0)
    pltpu.sync_copy(out_v, acc_sh.at[pl.ds(s * _RPT, _RPT)])
    plsc.subcore_barrier()

    g_base = wid * _EPW

    def _chunk(k, carry):
        base = g_base + k * _EK
        pltpu.sync_copy(flat_hbm.at[pl.ds(base, _EK)], idx_v)
        pltpu.sync_copy(ones_v.at[pl.ds(0, _EK)], acc_sh.at[idx_v], add=True)
        return carry
    lax.fori_loop(0, _NCH, _chunk, 0)
    plsc.subcore_barrier()

    pltpu.sync_copy(acc_sh.at[pl.ds(s * _RPT, _RPT)], out_v)
    pltpu.sync_copy(out_v, out_hbm.at[c, pl.ds(s * _RPT, _RPT)])


def _deg_partials(flat):
    kfn = pl.kernel(
        _deg_body,
        out_type=jax.ShapeDtypeStruct((_NC, _NP), jnp.float32),
        mesh=_mesh(),
        compiler_params=pltpu.CompilerParams(needs_layout_passes=False),
        scratch_types=[
            pltpu.VMEM((_EK,), jnp.int32),
            pltpu.VMEM((_RPT,), jnp.float32),
            pltpu.VMEM((_RPT,), jnp.float32),
            pltpu.VMEM_SHARED((_NP,), jnp.float32),
        ],
    )
    return kfn(flat)


def _dis_body(p_ref, dis_ref, inv_ref):
    deg = p_ref[0] + p_ref[1] + 1.0
    inv = 1.0 / deg
    inv_ref[...] = inv
    dis_ref[...] = jnp.sqrt(inv)


def _dis_kernel(degp):
    p3 = degp.reshape(_NC, _NP // D, D)
    return pl.pallas_call(
        _dis_body,
        out_shape=(jax.ShapeDtypeStruct((_NP // D, D), jnp.float32),
                   jax.ShapeDtypeStruct((_NP // D, D), jnp.float32)),
    )(p3)


def _vals_body(flat_hbm, dv_hbm, dis_hbm, out_hbm,
               dis_v, e0_v, e1_v, dv_v, val_v):
    c = lax.axis_index("c")
    s = lax.axis_index("s")
    wid = c * _NS + s
    n_edges = _VGP * _L
    base = wid * n_edges
    pltpu.sync_copy(dis_hbm, dis_v)
    pltpu.sync_copy(flat_hbm.at[pl.ds(base, n_edges)], e0_v)
    pltpu.sync_copy(flat_hbm.at[pl.ds(E + base, n_edges)], e1_v)
    pltpu.sync_copy(dv_hbm.at[pl.ds(base, n_edges)], dv_v)

    def _grp(g, carry):
        o = g * _L
        e0 = e0_v[pl.ds(o, _L)]
        e1 = e1_v[pl.ds(o, _L)]
        dv = dv_v[pl.ds(o, _L)]
        d0 = plsc.load_gather(dis_v, [e0])
        d1 = plsc.load_gather(dis_v, [e1])
        val_v[pl.ds(o, _L)] = jnp.exp(-dv * dv) * d0 * d1
        return carry
    lax.fori_loop(0, _VGP, _grp, 0)
    pltpu.sync_copy(val_v, out_hbm.at[pl.ds(base, n_edges)])


def _vals_kernel(flat_pad, dv_pad, dis_flat):
    kfn = pl.kernel(
        _vals_body,
        out_type=jax.ShapeDtypeStruct((_E_PAD,), jnp.float32),
        mesh=_mesh(),
        compiler_params=pltpu.CompilerParams(needs_layout_passes=False),
        scratch_types=[
            pltpu.VMEM((_NP,), jnp.float32),
            pltpu.VMEM((_VGP * _L,), jnp.int32),
            pltpu.VMEM((_VGP * _L,), jnp.int32),
            pltpu.VMEM((_VGP * _L,), jnp.float32),
            pltpu.VMEM((_VGP * _L,), jnp.float32),
        ],
    )
    return kfn(flat_pad, dv_pad, dis_flat)


def _agg_body(flat_hbm, vals_hbm, enc_hbm, out_hbm,
              ridx_v, cidx_v, vals_v, rows_v, stage_v, acc_sh, sem):
    c = lax.axis_index("c")
    s = lax.axis_index("s")
    wid = c * _NS + s
    zeros16 = jnp.zeros((_L,), jnp.float32)

    def _zs(j, carry):
        for d in range(D // _L):
            stage_v[j, pl.ds(d * _L, _L)] = zeros16
        return carry
    lax.fori_loop(0, 128, _zs, 0)
    for t in range(5):
        pltpu.sync_copy(stage_v, acc_sh.at[pl.ds(s * _RPT + t * 128, 128), :])
    plsc.subcore_barrier()

    g_base = wid * _EPW
    col_off = jnp.where(c == 0, E, -E)
    val_off = -c * E

    def _chunk(k, carry):
        base = g_base + k * _EK
        pltpu.sync_copy(flat_hbm.at[pl.ds(base, _EK)], ridx_v)
        pltpu.sync_copy(flat_hbm.at[pl.ds(base + col_off, _EK)], cidx_v)
        pltpu.sync_copy(vals_hbm.at[pl.ds(base + val_off, _EK)], vals_v)
        pltpu.async_copy(enc_hbm.at[cidx_v], rows_v, sem).wait()

        def _scale(j, cc):
            a16 = plsc.load_gather(vals_v, [jnp.zeros((_L,), jnp.int32) + j])
            for d in range(D // _L):
                rows_v[j, pl.ds(d * _L, _L)] = rows_v[j, pl.ds(d * _L, _L)] * a16
            return cc
        lax.fori_loop(0, _EK, _scale, 0)
        pltpu.sync_copy(rows_v, acc_sh.at[ridx_v], add=True)
        return carry
    lax.fori_loop(0, _NCH, _chunk, 0)
    plsc.subcore_barrier()

    for t in range(5):
        r0 = s * _RPT + t * 128
        pltpu.sync_copy(acc_sh.at[pl.ds(r0, 128), :], stage_v)
        pltpu.sync_copy(stage_v, out_hbm.at[c, pl.ds(r0, 128), :])


def _agg_kernel(flat_pad, vals_pad, enc):
    kfn = pl.kernel(
        _agg_body,
        out_type=jax.ShapeDtypeStruct((_NC, _NP, D), jnp.float32),
        mesh=_mesh(),
        compiler_params=pltpu.CompilerParams(needs_layout_passes=False),
        scratch_types=[
            pltpu.VMEM((_EK,), jnp.int32),
            pltpu.VMEM((_EK,), jnp.int32),
            pltpu.VMEM((_EK,), jnp.float32),
            pltpu.VMEM((_EK, D), jnp.float32),
            pltpu.VMEM((128, D), jnp.float32),
            pltpu.VMEM_SHARED((_NP, D), jnp.float32),
            pltpu.SemaphoreType.DMA,
        ],
    )
    return kfn(flat_pad, vals_pad, enc)


def _rowscale(x, col):
    return x * jnp.dot(col, jnp.ones((1, D), jnp.float32),
                       preferred_element_type=jnp.float32)


def _dense_body(p_ref, enc_ref, inv_ref, w_ref, b_ref, out_ref):
    side = p_ref[0] + p_ref[1] + _rowscale(enc_ref[...], inv_ref[...])
    h = jnp.dot(side, w_ref[...].T, preferred_element_type=jnp.float32)
    h = h + b_ref[...]
    h = jnp.where(h > 0, h, 0.01 * h)
    nrm = jnp.sqrt(jnp.sum(h * h, axis=1, keepdims=True))
    out_ref[...] = _rowscale(h, 1.0 / jnp.maximum(nrm, 1e-12))


def _dense_kernel(p, enc, inv_col, w, b):
    blk = 1000
    grid = N_POI // blk
    return pl.pallas_call(
        _dense_body,
        grid=(grid,),
        in_specs=[
            pl.BlockSpec((_NC, blk, D), lambda i: (0, i, 0)),
            pl.BlockSpec((blk, D), lambda i: (i, 0)),
            pl.BlockSpec((blk, 1), lambda i: (i, 0)),
            pl.BlockSpec((D, D), lambda i: (0, 0)),
            pl.BlockSpec((D,), lambda i: (0,)),
        ],
        out_specs=pl.BlockSpec((blk, D), lambda i: (i, 0)),
        out_shape=jax.ShapeDtypeStruct((N_POI, D), jnp.float32),
    )(p, enc, inv_col, w, b)


def _dense2_body(p_ref, enc_ref, inv_ref, w_ref, b_ref,
                 kw_ref, kb_ref, qw_ref, qb_ref,
                 out_ref, v2_ref, q2_ref):
    side = p_ref[0] + p_ref[1] + _rowscale(enc_ref[...], inv_ref[...])
    h = jnp.dot(side, w_ref[...].T, preferred_element_type=jnp.float32)
    h = h + b_ref[...]
    h = jnp.where(h > 0, h, 0.01 * h)
    nrm = jnp.sqrt(jnp.sum(h * h, axis=1, keepdims=True))
    enc2 = _rowscale(h, 1.0 / jnp.maximum(nrm, 1e-12))
    out_ref[...] = enc2
    v2_ref[...] = jnp.dot(enc2, kw_ref[...].T,
                          preferred_element_type=jnp.float32) + kb_ref[...]
    q2_ref[...] = jnp.dot(enc2, qw_ref[...].T,
                          preferred_element_type=jnp.float32) + qb_ref[...]


def _dense2_kernel(p, enc, inv_col, w, b, kw, kb, qw, qb):
    blk = 1000
    grid = N_POI // blk
    return pl.pallas_call(
        _dense2_body,
        grid=(grid,),
        in_specs=[
            pl.BlockSpec((_NC, blk, D), lambda i: (0, i, 0)),
            pl.BlockSpec((blk, D), lambda i: (i, 0)),
            pl.BlockSpec((blk, 1), lambda i: (i, 0)),
            pl.BlockSpec((D, D), lambda i: (0, 0)),
            pl.BlockSpec((D,), lambda i: (0,)),
            pl.BlockSpec((D, D), lambda i: (0, 0)),
            pl.BlockSpec((D,), lambda i: (0,)),
            pl.BlockSpec((D, D), lambda i: (0, 0)),
            pl.BlockSpec((D,), lambda i: (0,)),
        ],
        out_specs=(pl.BlockSpec((blk, D), lambda i: (i, 0)),
                   pl.BlockSpec((blk, D), lambda i: (i, 0)),
                   pl.BlockSpec((blk, D), lambda i: (i, 0))),
        out_shape=(jax.ShapeDtypeStruct((N_POI, D), jnp.float32),
                   jax.ShapeDtypeStruct((N_POI, D), jnp.float32),
                   jax.ShapeDtypeStruct((N_POI, D), jnp.float32)),
    )(p, enc, inv_col, w, b, kw, kb, qw, qb)


def _heads_body(aggr_ref, tar_ref, graph_ref,
                proj1_w_ref, proj1_b_ref, proj2_w_ref, proj2_b_ref,
                pred1_w_ref, pred1_b_ref, pred2_w_ref, pred2_b_ref,
                out1_ref, out2_ref):
    aggr = aggr_ref[...]
    tar = tar_ref[...]
    graph_enc = graph_ref[...]
    pred_in = jnp.concatenate([aggr, tar], axis=-1)
    h = pred_in @ pred1_w_ref[...].T + pred1_b_ref[...]
    h = jnp.where(h > 0, h, 0.01 * h)
    out2_ref[...] = h @ pred2_w_ref[...].T + pred2_b_ref[...]
    gh = graph_enc @ proj1_w_ref[...].T + proj1_b_ref[...]
    gh = jnp.where(gh > 0, gh, 0.01 * gh)
    out1_ref[...] = gh @ proj2_w_ref[...].T + proj2_b_ref[...]


def kernel(dist_edges, dist_vec, batch, poi, x, poi_table, gcn0_w, gcn0_b,
           gcn1_w, gcn1_b, K_w, K_b, Q_w, Q_b, V_w, V_b, proj1_w, proj1_b,
           proj2_w, proj2_b, pred1_w, pred1_b, pred2_w, pred2_b):
    flat = dist_edges.reshape(-1).astype(jnp.int32)
    flat_pad = jnp.pad(flat, (0, _E_PAD - E))
    dv_pad = jnp.pad(dist_vec, (0, _E_PAD - E))

    degp = _deg_partials(flat_pad)
    dis80, inv80 = _dis_kernel(degp)
    dis_flat = dis80.reshape(_NP)
    inv_col = inv80.reshape(_NP)[:N_POI].reshape(N_POI, 1)

    vals_pad = _vals_kernel(flat_pad, dv_pad, dis_flat)

    p0 = _agg_kernel(flat_pad, vals_pad, poi_table)
    enc1 = _dense_kernel(p0, poi_table, inv_col, gcn0_w, gcn0_b)
    p1 = _agg_kernel(flat_pad, vals_pad, enc1)
    enc2, V2, Q2 = _dense2_kernel(p1, enc1, inv_col, gcn1_w, gcn1_b,
                                  K_w, K_b, Q_w, Q_b)

    tar = enc2[poi]
    geo = enc2[x]
    seq_lens = jnp.bincount(batch, length=B)
    v = V2[x]
    q = Q2[poi]
    aw_logits = jnp.sum(v * q[batch], axis=-1)
    seg_max = jax.ops.segment_max(aw_logits, batch, num_segments=B)
    aw_exp = jnp.exp(aw_logits - seg_max[batch])
    aw_den = jax.ops.segment_sum(aw_exp, batch, num_segments=B)
    aw = aw_exp / aw_den[batch]
    seq_feat = jax.ops.segment_sum(v * aw[:, None], batch, num_segments=B)
    aggr = seq_feat @ V_w.T + V_b
    graph_enc = (jax.ops.segment_sum(geo, batch, num_segments=B)
                 / seq_lens[:, None].astype(jnp.float32))

    pred2_w_pad = jnp.pad(pred2_w, ((0, D - 1), (0, 0)))
    pred2_b_pad = jnp.pad(pred2_b, (0, D - 1))
    out1, out2_pad = pl.pallas_call(
        _heads_body,
        out_shape=(jax.ShapeDtypeStruct((B, D), jnp.float32),
                   jax.ShapeDtypeStruct((B, D), jnp.float32)),
    )(aggr, tar, graph_enc, proj1_w, proj1_b, proj2_w, proj2_b,
      pred1_w, pred1_b, pred2_w_pad, pred2_b_pad)
    return (out1, out2_pad[:, :1])
